# baseline (speedup 1.0000x reference)
import functools

import jax
import jax.numpy as jnp
from jax import lax
from jax.experimental import pallas as pl
from jax.experimental.pallas import tpu as pltpu
from jax.experimental.pallas import tpu_sc as plsc

B = 16384
C = 50
E = 64
SIGMA = 0.01

_NC = 2
_NS = 16
_NW = _NC * _NS
_BPW = B // _NW

_HBM = pltpu.MemorySpace.HBM
_NCH = 10


def _xcopy_body(xt_hbm, outp_hbm, xsems):
    cch = C // _NCH
    copies = []
    for i in range(_NCH):
        c = pltpu.make_async_copy(
            xt_hbm.at[pl.ds(i * cch, cch)],
            outp_hbm.at[pl.ds(1 + i * cch, cch)],
            xsems.at[i],
        )
        c.start()
        copies.append(c)
    for c in copies:
        c.wait()


def _tc_xcopy(xt):
    return pl.pallas_call(
        _xcopy_body,
        in_specs=[pl.BlockSpec(memory_space=_HBM)],
        out_specs=pl.BlockSpec(memory_space=_HBM),
        out_shape=jax.ShapeDtypeStruct((C + 1, E, B), jnp.float32),
        scratch_shapes=[pltpu.SemaphoreType.DMA((_NCH,))],
    )(xt)


_sc_mesh = plsc.VectorSubcoreMesh(core_axis_name="c", subcore_axis_name="s")

_LCH = 512
_NFULL = 1953
_TAIL = 1000000 - _NFULL * _LCH
_CAP = 640
_SPASS = 4096


@functools.partial(
    pl.kernel,
    mesh=_sc_mesh,
    compiler_params=pltpu.CompilerParams(needs_layout_passes=False),
    out_type=jax.ShapeDtypeStruct((B + 8, 128), jnp.float32),
    scratch_types=[
        pltpu.VMEM((_SPASS,), jnp.int32),
        pltpu.VMEM((_CAP,), jnp.int32),
        pltpu.VMEM((_CAP // 128, 128), jnp.int32),
        pltpu.VMEM((E, _LCH), jnp.float32),
        pltpu.VMEM((_CAP,), jnp.int32),
        pltpu.VMEM((_CAP,), jnp.int32),
        pltpu.VMEM((_CAP, 128), jnp.float32),
        pltpu.SemaphoreType.DMA,
    ],
)
def _sc_gather(subject_hbm, tablet_hbm, tail128_hbm, emb2_hbm, subbuf, mysubs,
               mypos, chunkbuf, clane, cslot, rowbuf, sem):
    wid = lax.axis_index("s") * _NC + lax.axis_index("c")
    base = jnp.where(wid == 0, 0, 61 * wid + 1)
    ncheff = jnp.where((wid == 0) | (wid == _NW - 1), 62, 61)
    i16 = lax.iota(jnp.int32, 16)

    for g in range(_CAP // 16):
        mysubs[pl.ds(g * 16, 16)] = jnp.full((16,), jnp.int32(0x40000000))
    for j in range(_CAP // 128):
        for g in range(8):
            mypos[j, pl.ds(g * 16, 16)] = jnp.full((16,), jnp.int32(B))

    def sel_pass(p, cnt0):
        pltpu.sync_copy(subject_hbm.at[pl.ds(p * _SPASS, _SPASS)], subbuf)

        def sel(i, cnt):
            sv = subbuf[pl.ds(i * 16, 16)]
            c = lax.shift_right_logical(sv, 9)
            mine = (c >= base) & (c < base + ncheff)
            minei = mine.astype(jnp.int32)
            pos_in = jnp.minimum(cnt + plsc.cumsum(minei) - 1, _CAP - 1)
            posv = p * _SPASS + i * 16 + i16
            plsc.store_scatter(mysubs, [pos_in], sv, mask=mine)
            plsc.store_scatter(
                mypos,
                [lax.shift_right_logical(pos_in, 7), pos_in & 127],
                posv,
                mask=mine,
            )
            return cnt + jnp.sum(minei)

        return lax.fori_loop(0, _SPASS // 16, sel, cnt0)

    lax.fori_loop(0, B // _SPASS, sel_pass, jnp.int32(0))

    def chunk_step(ci, _):
        chunkidx = base + ci
        lane0 = pl.multiple_of(chunkidx * _LCH, _LCH)

        @pl.when(chunkidx < _NFULL)
        def _fetch_full():
            pltpu.sync_copy(tablet_hbm.at[:, pl.ds(lane0, _LCH)], chunkbuf)

        @pl.when(chunkidx == _NFULL)
        def _fetch_tail():
            pltpu.sync_copy(tail128_hbm, chunkbuf.at[:, pl.ds(0, 128)])

        def comp(g, m):
            sv = mysubs[pl.ds(g * 16, 16)]
            mk = lax.shift_right_logical(sv, 9) == chunkidx
            mki = mk.astype(jnp.int32)
            pi = jnp.minimum(m + plsc.cumsum(mki) - 1, _CAP - 1)
            plsc.store_scatter(clane, [pi], sv - lane0, mask=mk)
            plsc.store_scatter(cslot, [pi], g * 16 + i16, mask=mk)
            return m + jnp.sum(mki)

        m = lax.fori_loop(0, _CAP // 16, comp, jnp.int32(0))

        def ext(gi, _):
            offs = clane[pl.ds(gi * 16, 16)]
            slots = cslot[pl.ds(gi * 16, 16)]
            valid = (gi * 16 + i16) < m
            for e in range(E):
                erow = jnp.full((16,), jnp.int32(e))
                vals = plsc.load_gather(chunkbuf, [erow, offs], mask=valid)
                plsc.store_scatter(rowbuf, [slots, erow], vals, mask=valid)
            return 0

        lax.fori_loop(0, (m + 15) // 16, ext, 0)
        return 0

    lax.fori_loop(0, ncheff, chunk_step, 0)

    copies = [
        pltpu.async_copy(
            rowbuf.at[pl.ds(j * 128, 128)], emb2_hbm.at[mypos.at[j]], sem
        )
        for j in range(_CAP // 128)
    ]
    for c in copies:
        c.wait()


def _fill_body(outp_hbm, emb_hbm, noiset_hbm, out_hbm, ebuf, nbuf, tbuf, sem):
    e_in = pltpu.make_async_copy(emb_hbm.at[pl.ds(0, B)], ebuf, sem)
    e_in.start()
    n_in = pltpu.make_async_copy(noiset_hbm, nbuf, sem)
    n_in.start()
    e_in.wait()
    n_in.wait()
    tbuf[...] = jnp.transpose(ebuf[:, :E], (1, 0)) + nbuf[...]
    r_out = pltpu.make_async_copy(tbuf, out_hbm.at[0], sem)
    r_out.start()
    r_out.wait()


def _tc_fill(outp, emb, noiset):
    return pl.pallas_call(
        _fill_body,
        in_specs=[
            pl.BlockSpec(memory_space=_HBM),
            pl.BlockSpec(memory_space=_HBM),
            pl.BlockSpec(memory_space=_HBM),
        ],
        out_specs=pl.BlockSpec(memory_space=_HBM),
        out_shape=jax.ShapeDtypeStruct((C + 1, E, B), jnp.float32),
        input_output_aliases={0: 0},
        scratch_shapes=[
            pltpu.VMEM((B, 128), jnp.float32),
            pltpu.VMEM((E, B), jnp.float32),
            pltpu.VMEM((E, B), jnp.float32),
            pltpu.SemaphoreType.DMA,
        ],
    )(outp, emb, noiset)


def kernel(x, subject, table):
    noise = jax.random.normal(jax.random.key(42), (B, 1, E), dtype=jnp.float32)
    noiset = jnp.transpose((noise * SIGMA).reshape(B, E))
    xt = jnp.transpose(x, (1, 2, 0))
    tablet = jnp.transpose(table)
    tail128 = jnp.pad(tablet[:, _NFULL * _LCH:], ((0, 0), (0, 128 - _TAIL)))
    outp = _tc_xcopy(xt)
    emb2 = _sc_gather(subject.astype(jnp.int32), tablet, tail128)
    out = _tc_fill(outp, emb2, noiset)
    return jnp.transpose(out, (2, 0, 1))


# speedup vs baseline: 14.4140x; 14.4140x over previous
import functools

import jax
import jax.numpy as jnp
from jax import lax
from jax.experimental import pallas as pl
from jax.experimental.pallas import tpu as pltpu
from jax.experimental.pallas import tpu_sc as plsc

B = 16384
C = 50
E = 64
SIGMA = 0.01

_NC = 2
_NS = 16
_NW = _NC * _NS
_BPW = B // _NW

_HBM = pltpu.MemorySpace.HBM
_NCH = 10


def _xcopy_body(x_ref, out_ref):
    out_ref[...] = x_ref[...]


def _tc_xcopy(xt):
    return pl.pallas_call(
        _xcopy_body,
        grid=(C,),
        in_specs=[pl.BlockSpec((1, E, B), lambda c: (c, 0, 0))],
        out_specs=pl.BlockSpec((1, E, B), lambda c: (c + 1, 0, 0)),
        out_shape=jax.ShapeDtypeStruct((C + 1, E, B), jnp.float32),
        compiler_params=pltpu.CompilerParams(
            dimension_semantics=("arbitrary",),
        ),
    )(xt)


_sc_mesh = plsc.VectorSubcoreMesh(core_axis_name="c", subcore_axis_name="s")

_LCH = 512
_NFULL = 1953
_TAIL = 1000000 - _NFULL * _LCH
_CAP = 640
_SPASS = 4096


@functools.partial(
    pl.kernel,
    mesh=_sc_mesh,
    compiler_params=pltpu.CompilerParams(needs_layout_passes=False),
    out_type=jax.ShapeDtypeStruct((B + 8, 128), jnp.float32),
    scratch_types=[
        pltpu.VMEM((_SPASS,), jnp.int32),
        pltpu.VMEM((_CAP,), jnp.int32),
        pltpu.VMEM((_CAP // 128, 128), jnp.int32),
        pltpu.VMEM((E, _LCH), jnp.float32),
        pltpu.VMEM((_CAP,), jnp.int32),
        pltpu.VMEM((_CAP,), jnp.int32),
        pltpu.VMEM((_CAP, 128), jnp.float32),
        pltpu.SemaphoreType.DMA,
    ],
)
def _sc_gather(subject_hbm, tablet_hbm, tail128_hbm, emb2_hbm, subbuf, mysubs,
               mypos, chunkbuf, clane, cslot, rowbuf, sem):
    wid = lax.axis_index("s") * _NC + lax.axis_index("c")
    base = jnp.where(wid == 0, 0, 61 * wid + 1)
    ncheff = jnp.where((wid == 0) | (wid == _NW - 1), 62, 61)
    i16 = lax.iota(jnp.int32, 16)

    for g in range(_CAP // 16):
        mysubs[pl.ds(g * 16, 16)] = jnp.full((16,), jnp.int32(0x40000000))
    for j in range(_CAP // 128):
        for g in range(8):
            mypos[j, pl.ds(g * 16, 16)] = jnp.full((16,), jnp.int32(B))

    def sel_pass(p, cnt0):
        pltpu.sync_copy(subject_hbm.at[pl.ds(p * _SPASS, _SPASS)], subbuf)

        def sel(i, cnt):
            sv = subbuf[pl.ds(i * 16, 16)]
            c = lax.shift_right_logical(sv, 9)
            mine = (c >= base) & (c < base + ncheff)
            minei = mine.astype(jnp.int32)
            pos_in = jnp.minimum(cnt + plsc.cumsum(minei) - 1, _CAP - 1)
            posv = p * _SPASS + i * 16 + i16
            plsc.store_scatter(mysubs, [pos_in], sv, mask=mine)
            plsc.store_scatter(
                mypos,
                [lax.shift_right_logical(pos_in, 7), pos_in & 127],
                posv,
                mask=mine,
            )
            return cnt + jnp.sum(minei)

        return lax.fori_loop(0, _SPASS // 16, sel, cnt0)

    lax.fori_loop(0, B // _SPASS, sel_pass, jnp.int32(0))

    def chunk_step(ci, _):
        chunkidx = base + ci
        lane0 = pl.multiple_of(chunkidx * _LCH, _LCH)

        @pl.when(chunkidx < _NFULL)
        def _fetch_full():
            pltpu.sync_copy(tablet_hbm.at[:, pl.ds(lane0, _LCH)], chunkbuf)

        @pl.when(chunkidx == _NFULL)
        def _fetch_tail():
            pltpu.sync_copy(tail128_hbm, chunkbuf.at[:, pl.ds(0, 128)])

        def comp(g, m):
            sv = mysubs[pl.ds(g * 16, 16)]
            mk = lax.shift_right_logical(sv, 9) == chunkidx
            mki = mk.astype(jnp.int32)
            pi = jnp.minimum(m + plsc.cumsum(mki) - 1, _CAP - 1)
            plsc.store_scatter(clane, [pi], sv - lane0, mask=mk)
            plsc.store_scatter(cslot, [pi], g * 16 + i16, mask=mk)
            return m + jnp.sum(mki)

        m = lax.fori_loop(0, _CAP // 16, comp, jnp.int32(0))

        def ext(gi, _):
            offs = clane[pl.ds(gi * 16, 16)]
            slots = cslot[pl.ds(gi * 16, 16)]
            valid = (gi * 16 + i16) < m
            for e in range(E):
                erow = jnp.full((16,), jnp.int32(e))
                vals = plsc.load_gather(chunkbuf, [erow, offs], mask=valid)
                plsc.store_scatter(rowbuf, [slots, erow], vals, mask=valid)
            return 0

        lax.fori_loop(0, (m + 15) // 16, ext, 0)
        return 0

    lax.fori_loop(0, ncheff, chunk_step, 0)

    copies = [
        pltpu.async_copy(
            rowbuf.at[pl.ds(j * 128, 128)], emb2_hbm.at[mypos.at[j]], sem
        )
        for j in range(_CAP // 128)
    ]
    for c in copies:
        c.wait()


def _fill_body(outp_hbm, emb_hbm, noiset_hbm, out_hbm, ebuf, nbuf, tbuf, sem):
    e_in = pltpu.make_async_copy(emb_hbm.at[pl.ds(0, B)], ebuf, sem)
    e_in.start()
    n_in = pltpu.make_async_copy(noiset_hbm, nbuf, sem)
    n_in.start()
    e_in.wait()
    n_in.wait()
    tbuf[...] = jnp.transpose(ebuf[:, :E], (1, 0)) + nbuf[...]
    r_out = pltpu.make_async_copy(tbuf, out_hbm.at[0], sem)
    r_out.start()
    r_out.wait()


def _tc_fill(outp, emb, noiset):
    return pl.pallas_call(
        _fill_body,
        in_specs=[
            pl.BlockSpec(memory_space=_HBM),
            pl.BlockSpec(memory_space=_HBM),
            pl.BlockSpec(memory_space=_HBM),
        ],
        out_specs=pl.BlockSpec(memory_space=_HBM),
        out_shape=jax.ShapeDtypeStruct((C + 1, E, B), jnp.float32),
        input_output_aliases={0: 0},
        scratch_shapes=[
            pltpu.VMEM((B, 128), jnp.float32),
            pltpu.VMEM((E, B), jnp.float32),
            pltpu.VMEM((E, B), jnp.float32),
            pltpu.SemaphoreType.DMA,
        ],
    )(outp, emb, noiset)


def kernel(x, subject, table):
    noise = jax.random.normal(jax.random.key(42), (B, 1, E), dtype=jnp.float32)
    noiset = jnp.transpose((noise * SIGMA).reshape(B, E))
    xt = jnp.transpose(x, (1, 2, 0))
    tablet = jnp.transpose(table)
    tail128 = jnp.pad(tablet[:, _NFULL * _LCH:], ((0, 0), (0, 128 - _TAIL)))
    outp = _tc_xcopy(xt)
    emb2 = _sc_gather(subject.astype(jnp.int32), tablet, tail128)
    out = _tc_fill(outp, emb2, noiset)
    return jnp.transpose(out, (2, 0, 1))


# speedup vs baseline: 14.4542x; 1.0028x over previous
import functools

import jax
import jax.numpy as jnp
from jax import lax
from jax.experimental import pallas as pl
from jax.experimental.pallas import tpu as pltpu
from jax.experimental.pallas import tpu_sc as plsc

B = 16384
C = 50
E = 64
SIGMA = 0.01

_NC = 2
_NS = 16
_NW = _NC * _NS
_BPW = B // _NW

_HBM = pltpu.MemorySpace.HBM
_NCH = 10


def _xcopy_body(x_ref, out_ref):
    out_ref[...] = x_ref[...]


def _tc_xcopy(xt):
    return pl.pallas_call(
        _xcopy_body,
        grid=(C,),
        in_specs=[pl.BlockSpec((1, E, B), lambda c: (c, 0, 0))],
        out_specs=pl.BlockSpec((1, E, B), lambda c: (c + 1, 0, 0)),
        out_shape=jax.ShapeDtypeStruct((C + 1, E, B), jnp.float32),
        compiler_params=pltpu.CompilerParams(
            dimension_semantics=("arbitrary",),
        ),
    )(xt)


_sc_mesh = plsc.VectorSubcoreMesh(core_axis_name="c", subcore_axis_name="s")

_LCH = 512
_NFULL = 1953
_TAIL = 1000000 - _NFULL * _LCH
_CAP = 640
_SPASS = 4096


@functools.partial(
    pl.kernel,
    mesh=_sc_mesh,
    compiler_params=pltpu.CompilerParams(needs_layout_passes=False),
    out_type=jax.ShapeDtypeStruct((B + 8, 128), jnp.float32),
    scratch_types=[
        pltpu.VMEM((_SPASS,), jnp.int32),
        pltpu.VMEM((_CAP,), jnp.int32),
        pltpu.VMEM((_CAP,), jnp.int32),
        pltpu.VMEM((_CAP // 128, 128), jnp.int32),
        pltpu.VMEM((E, _LCH), jnp.float32),
        pltpu.VMEM((_CAP,), jnp.int32),
        pltpu.VMEM((_CAP,), jnp.int32),
        pltpu.VMEM((_CAP, 128), jnp.float32),
        pltpu.SemaphoreType.DMA,
    ],
)
def _sc_gather(subject_hbm, tablet_hbm, tail128_hbm, emb2_hbm, subbuf, mysubs,
               myposl, mypos, chunkbuf, clane, cslot, rowbuf, sem):
    wid = lax.axis_index("s") * _NC + lax.axis_index("c")
    base = jnp.where(wid == 0, 0, 61 * wid + 1)
    ncheff = jnp.where((wid == 0) | (wid == _NW - 1), 62, 61)
    i16 = lax.iota(jnp.int32, 16)

    for g in range(_CAP // 16):
        mysubs[pl.ds(g * 16, 16)] = jnp.full((16,), jnp.int32(0x40000000))
        myposl[pl.ds(g * 16, 16)] = jnp.full((16,), jnp.int32(B))

    def sel_pass(p, cnt0):
        pltpu.sync_copy(subject_hbm.at[pl.ds(p * _SPASS, _SPASS)], subbuf)

        def sel(i, cnt):
            sv = subbuf[pl.ds(i * 16, 16)]
            c = lax.shift_right_logical(sv, 9)
            mine = (c >= base) & (c < base + ncheff)
            posv = p * _SPASS + i * 16 + i16
            off = jnp.minimum(cnt, _CAP - 16)
            plsc.store_compressed(mysubs.at[pl.ds(off, 16)], sv, mask=mine)
            plsc.store_compressed(myposl.at[pl.ds(off, 16)], posv, mask=mine)
            n16 = plsc.all_reduce_population_count(mine)
            return cnt + n16[0]

        return lax.fori_loop(0, _SPASS // 16, sel, cnt0)

    lax.fori_loop(0, B // _SPASS, sel_pass, jnp.int32(0))

    for j in range(_CAP // 128):
        for g in range(8):
            mypos[j, pl.ds(g * 16, 16)] = myposl[pl.ds(j * 128 + g * 16, 16)]

    def chunk_step(ci, _):
        chunkidx = base + ci
        lane0 = pl.multiple_of(chunkidx * _LCH, _LCH)

        @pl.when(chunkidx < _NFULL)
        def _fetch_full():
            pltpu.sync_copy(tablet_hbm.at[:, pl.ds(lane0, _LCH)], chunkbuf)

        @pl.when(chunkidx == _NFULL)
        def _fetch_tail():
            pltpu.sync_copy(tail128_hbm, chunkbuf.at[:, pl.ds(0, 128)])

        def comp(g, m):
            sv = mysubs[pl.ds(g * 16, 16)]
            mk = lax.shift_right_logical(sv, 9) == chunkidx
            mki = mk.astype(jnp.int32)
            pi = jnp.minimum(m + plsc.cumsum(mki) - 1, _CAP - 1)
            plsc.store_scatter(clane, [pi], sv - lane0, mask=mk)
            plsc.store_scatter(cslot, [pi], g * 16 + i16, mask=mk)
            return m + jnp.sum(mki)

        m = lax.fori_loop(0, _CAP // 16, comp, jnp.int32(0))

        def ext(gi, _):
            offs = clane[pl.ds(gi * 16, 16)]
            slots = cslot[pl.ds(gi * 16, 16)]
            valid = (gi * 16 + i16) < m
            for e in range(E):
                erow = jnp.full((16,), jnp.int32(e))
                vals = plsc.load_gather(chunkbuf, [erow, offs], mask=valid)
                plsc.store_scatter(rowbuf, [slots, erow], vals, mask=valid)
            return 0

        lax.fori_loop(0, (m + 15) // 16, ext, 0)
        return 0

    lax.fori_loop(0, ncheff, chunk_step, 0)

    copies = [
        pltpu.async_copy(
            rowbuf.at[pl.ds(j * 128, 128)], emb2_hbm.at[mypos.at[j]], sem
        )
        for j in range(_CAP // 128)
    ]
    for c in copies:
        c.wait()


def _fill_body(outp_hbm, emb_hbm, noiset_hbm, out_hbm, ebuf, nbuf, tbuf, sem):
    e_in = pltpu.make_async_copy(emb_hbm.at[pl.ds(0, B)], ebuf, sem)
    e_in.start()
    n_in = pltpu.make_async_copy(noiset_hbm, nbuf, sem)
    n_in.start()
    e_in.wait()
    n_in.wait()
    tbuf[...] = jnp.transpose(ebuf[:, :E], (1, 0)) + nbuf[...]
    r_out = pltpu.make_async_copy(tbuf, out_hbm.at[0], sem)
    r_out.start()
    r_out.wait()


def _tc_fill(outp, emb, noiset):
    return pl.pallas_call(
        _fill_body,
        in_specs=[
            pl.BlockSpec(memory_space=_HBM),
            pl.BlockSpec(memory_space=_HBM),
            pl.BlockSpec(memory_space=_HBM),
        ],
        out_specs=pl.BlockSpec(memory_space=_HBM),
        out_shape=jax.ShapeDtypeStruct((C + 1, E, B), jnp.float32),
        input_output_aliases={0: 0},
        scratch_shapes=[
            pltpu.VMEM((B, 128), jnp.float32),
            pltpu.VMEM((E, B), jnp.float32),
            pltpu.VMEM((E, B), jnp.float32),
            pltpu.SemaphoreType.DMA,
        ],
    )(outp, emb, noiset)


def kernel(x, subject, table):
    noise = jax.random.normal(jax.random.key(42), (B, 1, E), dtype=jnp.float32)
    noiset = jnp.transpose((noise * SIGMA).reshape(B, E))
    xt = jnp.transpose(x, (1, 2, 0))
    tablet = jnp.transpose(table)
    tail128 = jnp.pad(tablet[:, _NFULL * _LCH:], ((0, 0), (0, 128 - _TAIL)))
    outp = _tc_xcopy(xt)
    emb2 = _sc_gather(subject.astype(jnp.int32), tablet, tail128)
    out = _tc_fill(outp, emb2, noiset)
    return jnp.transpose(out, (2, 0, 1))


# speedup vs baseline: 14.6647x; 1.0146x over previous
import functools

import jax
import jax.numpy as jnp
from jax import lax
from jax.experimental import pallas as pl
from jax.experimental.pallas import tpu as pltpu
from jax.experimental.pallas import tpu_sc as plsc

B = 16384
C = 50
E = 64
SIGMA = 0.01

_NC = 2
_NS = 16
_NW = _NC * _NS
_BPW = B // _NW

_HBM = pltpu.MemorySpace.HBM
_NCH = 10


def _xcopy_body(x_ref, out_ref):
    out_ref[...] = x_ref[...]


def _tc_xcopy(xt):
    return pl.pallas_call(
        _xcopy_body,
        grid=(C,),
        in_specs=[pl.BlockSpec((1, E, B), lambda c: (c, 0, 0))],
        out_specs=pl.BlockSpec((1, E, B), lambda c: (c + 1, 0, 0)),
        out_shape=jax.ShapeDtypeStruct((C + 1, E, B), jnp.float32),
        compiler_params=pltpu.CompilerParams(
            dimension_semantics=("arbitrary",),
        ),
    )(xt)


_sc_mesh = plsc.VectorSubcoreMesh(core_axis_name="c", subcore_axis_name="s")

_LCH = 512
_NFULL = 1953
_TAIL = 1000000 - _NFULL * _LCH
_CAP = 640
_SPASS = 4096


@functools.partial(
    pl.kernel,
    mesh=_sc_mesh,
    compiler_params=pltpu.CompilerParams(needs_layout_passes=False),
    out_type=jax.ShapeDtypeStruct((B + 8, 128), jnp.float32),
    scratch_types=[
        pltpu.VMEM((_SPASS,), jnp.int32),
        pltpu.VMEM((_CAP,), jnp.int32),
        pltpu.VMEM((_CAP,), jnp.int32),
        pltpu.VMEM((_CAP // 128, 128), jnp.int32),
        pltpu.VMEM((E, _LCH), jnp.float32),
        pltpu.VMEM((_CAP,), jnp.int32),
        pltpu.VMEM((_CAP,), jnp.int32),
        pltpu.VMEM((_CAP, 128), jnp.float32),
        pltpu.SemaphoreType.DMA,
    ],
)
def _sc_gather(subject_hbm, tablet_hbm, tail128_hbm, emb2_hbm, subbuf, mysubs,
               myposl, mypos, chunkbuf, clane, cslot, rowbuf, sem):
    wid = lax.axis_index("s") * _NC + lax.axis_index("c")
    base = jnp.where(wid == 0, 0, 61 * wid + 1)
    ncheff = jnp.where((wid == 0) | (wid == _NW - 1), 62, 61)
    i16 = lax.iota(jnp.int32, 16)

    for g in range(_CAP // 16):
        mysubs[pl.ds(g * 16, 16)] = jnp.full((16,), jnp.int32(0x40000000))
        myposl[pl.ds(g * 16, 16)] = jnp.full((16,), jnp.int32(B))

    def sel_pass(p, cnt0):
        pltpu.sync_copy(subject_hbm.at[pl.ds(p * _SPASS, _SPASS)], subbuf)

        @plsc.parallel_loop(0, _SPASS, step=16, unroll=4, carry=cnt0)
        def cnt1(i, cnt):
            sv = subbuf[pl.ds(i, 16)]
            c = lax.shift_right_logical(sv, 9)
            mine = (c >= base) & (c < base + ncheff)
            posv = p * _SPASS + i + i16
            off = jnp.minimum(cnt, _CAP - 16)
            plsc.store_compressed(mysubs.at[pl.ds(off, 16)], sv, mask=mine)
            plsc.store_compressed(myposl.at[pl.ds(off, 16)], posv, mask=mine)
            n16 = plsc.all_reduce_population_count(mine)
            return cnt + n16[0]

        return cnt1

    lax.fori_loop(0, B // _SPASS, sel_pass, jnp.int32(0))

    for j in range(_CAP // 128):
        for g in range(8):
            mypos[j, pl.ds(g * 16, 16)] = myposl[pl.ds(j * 128 + g * 16, 16)]

    def chunk_step(ci, _):
        chunkidx = base + ci
        lane0 = pl.multiple_of(chunkidx * _LCH, _LCH)

        @pl.when(chunkidx < _NFULL)
        def _fetch_full():
            pltpu.sync_copy(tablet_hbm.at[:, pl.ds(lane0, _LCH)], chunkbuf)

        @pl.when(chunkidx == _NFULL)
        def _fetch_tail():
            pltpu.sync_copy(tail128_hbm, chunkbuf.at[:, pl.ds(0, 128)])

        def comp(g, m):
            sv = mysubs[pl.ds(g * 16, 16)]
            mk = lax.shift_right_logical(sv, 9) == chunkidx
            mki = mk.astype(jnp.int32)
            pi = jnp.minimum(m + plsc.cumsum(mki) - 1, _CAP - 1)
            plsc.store_scatter(clane, [pi], sv - lane0, mask=mk)
            plsc.store_scatter(cslot, [pi], g * 16 + i16, mask=mk)
            return m + jnp.sum(mki)

        m = lax.fori_loop(0, _CAP // 16, comp, jnp.int32(0))

        def ext(gi, _):
            offs = clane[pl.ds(gi * 16, 16)]
            slots = cslot[pl.ds(gi * 16, 16)]
            valid = (gi * 16 + i16) < m
            for e in range(E):
                erow = jnp.full((16,), jnp.int32(e))
                vals = plsc.load_gather(chunkbuf, [erow, offs], mask=valid)
                plsc.store_scatter(rowbuf, [slots, erow], vals, mask=valid)
            return 0

        lax.fori_loop(0, (m + 15) // 16, ext, 0)
        return 0

    lax.fori_loop(0, ncheff, chunk_step, 0)

    copies = [
        pltpu.async_copy(
            rowbuf.at[pl.ds(j * 128, 128)], emb2_hbm.at[mypos.at[j]], sem
        )
        for j in range(_CAP // 128)
    ]
    for c in copies:
        c.wait()


def _fill_body(outp_hbm, emb_hbm, noiset_hbm, out_hbm, ebuf, nbuf, tbuf, sem):
    e_in = pltpu.make_async_copy(emb_hbm.at[pl.ds(0, B)], ebuf, sem)
    e_in.start()
    n_in = pltpu.make_async_copy(noiset_hbm, nbuf, sem)
    n_in.start()
    e_in.wait()
    n_in.wait()
    tbuf[...] = jnp.transpose(ebuf[:, :E], (1, 0)) + nbuf[...]
    r_out = pltpu.make_async_copy(tbuf, out_hbm.at[0], sem)
    r_out.start()
    r_out.wait()


def _tc_fill(outp, emb, noiset):
    return pl.pallas_call(
        _fill_body,
        in_specs=[
            pl.BlockSpec(memory_space=_HBM),
            pl.BlockSpec(memory_space=_HBM),
            pl.BlockSpec(memory_space=_HBM),
        ],
        out_specs=pl.BlockSpec(memory_space=_HBM),
        out_shape=jax.ShapeDtypeStruct((C + 1, E, B), jnp.float32),
        input_output_aliases={0: 0},
        scratch_shapes=[
            pltpu.VMEM((B, 128), jnp.float32),
            pltpu.VMEM((E, B), jnp.float32),
            pltpu.VMEM((E, B), jnp.float32),
            pltpu.SemaphoreType.DMA,
        ],
    )(outp, emb, noiset)


def kernel(x, subject, table):
    noise = jax.random.normal(jax.random.key(42), (B, 1, E), dtype=jnp.float32)
    noiset = jnp.transpose((noise * SIGMA).reshape(B, E))
    xt = jnp.transpose(x, (1, 2, 0))
    tablet = jnp.transpose(table)
    tail128 = jnp.pad(tablet[:, _NFULL * _LCH:], ((0, 0), (0, 128 - _TAIL)))
    outp = _tc_xcopy(xt)
    emb2 = _sc_gather(subject.astype(jnp.int32), tablet, tail128)
    out = _tc_fill(outp, emb2, noiset)
    return jnp.transpose(out, (2, 0, 1))
